# baseline (speedup 1.0000x reference)
import functools

import jax
import jax.numpy as jnp
from jax import lax
from jax.experimental import pallas as pl
from jax.experimental.pallas import tpu as pltpu


def _sigmoid(a):
    return 0.5 * jnp.tanh(0.5 * a) + 0.5


def _gru_chunk_kernel(x_ref, ta_ref, tb_ref, wihx_ref, wiht_ref, whh_ref,
                      bgi_ref, bhn_ref, wl_ref, bl_ref, zm_ref, zv_ref,
                      h_ref, do_ref, wrz_ref, wn_ref, whn_ref, *, nc):
    c = pl.program_id(1)
    CS, BB, Din = x_ref.shape
    HP = whh_ref.shape[0]
    G3 = whh_ref.shape[1]
    H2 = 2 * HP
    KD = whn_ref.shape[0] - HP

    @pl.when(c == 0)
    def _init():
        h_ref[...] = jnp.zeros_like(h_ref)
        wrz_ref[...] = jnp.concatenate(
            [wihx_ref[:, :H2], whh_ref[:, :H2]], axis=0)
        wn_ref[...] = jnp.concatenate(
            [wihx_ref[:, H2:], wiht_ref[:, H2:], bgi_ref[:, H2:],
             jnp.zeros((KD - 2, HP), jnp.float32)], axis=0)
        whn_ref[...] = jnp.concatenate(
            [whh_ref[:, H2:], jnp.zeros((1, HP), jnp.float32),
             bhn_ref[...], jnp.zeros((KD - 2, HP), jnp.float32)], axis=0)

    ta = ta_ref[...]
    d_rest = ta[:-1] - ta[1:]
    mask = jnp.where(c == nc - 1, 0.0, 1.0)
    d0 = (tb_ref[0:1] - ta[0:1]) * mask
    dt = jnp.concatenate([d0, d_rest], axis=0)
    ones = jnp.ones((CS, BB, 1), jnp.float32)
    zeros = jnp.zeros((CS, BB, KD - 2), jnp.float32)
    do_ref[...] = jnp.concatenate([dt, ones, zeros], axis=2)

    HB = BB // 2
    wt_rz = wiht_ref[:, :H2]
    bg_rz = bgi_ref[:, :H2]

    def half(xh, dh, h):
        arz = jnp.dot(jnp.concatenate([xh, h], axis=1), wrz_ref[...],
                      preferred_element_type=jnp.float32)
        arz = arz + (dh[:, 0:1] * wt_rz + bg_rz)
        gin = jnp.dot(jnp.concatenate([xh, dh], axis=1), wn_ref[...],
                      preferred_element_type=jnp.float32)
        ghn = jnp.dot(jnp.concatenate([h, dh], axis=1), whn_ref[...],
                      preferred_element_type=jnp.float32)
        r = _sigmoid(arz[:, :HP])
        z = _sigmoid(arz[:, HP:])
        n = jnp.tanh(gin + r * ghn)
        return (1.0 - z) * n + z * h

    def step(i, carry):
        h0, h1 = carry
        s = CS - 1 - i
        xs = x_ref[s]
        ds = do_ref[s]
        return (half(xs[:HB], ds[:HB], h0), half(xs[HB:], ds[HB:], h1))

    h0, h1 = lax.fori_loop(0, CS, step, (h_ref[:HB], h_ref[HB:]),
                           unroll=True)
    h_ref[:HB] = h0
    h_ref[HB:] = h1

    @pl.when(c == nc - 1)
    def _head():
        L = zm_ref.shape[1]
        z0 = (jnp.dot(h_ref[...], wl_ref[...], preferred_element_type=jnp.float32)
              + bl_ref[...])
        zm_ref[...] = z0[:, :L]
        zv_ref[...] = z0[:, L:]


def kernel(x, t, wihx, wiht, whh, bgi, bhn, wl, bl):
    S, B, Din = x.shape
    HP = whh.shape[0]
    G3 = whh.shape[1]
    L2 = wl.shape[1]
    latent_dim = L2 // 2

    x = x.astype(jnp.float32)
    t = t.astype(jnp.float32)

    B_pad = max(16, ((B + 15) // 16) * 16)
    block_b = min(B_pad, 256)
    block_b = max(16, (block_b // 16) * 16)
    B_pad = ((B_pad + block_b - 1) // block_b) * block_b
    if B_pad != B:
        x = jnp.pad(x, ((0, 0), (0, B_pad - B), (0, 0)))
        t = jnp.pad(t, ((0, 0), (0, B_pad - B), (0, 0)))

    cs = 1
    for cand in range(min(S, 50), 0, -1):
        if S % cand == 0:
            cs = cand
            break
    nc = S // cs

    grid = (B_pad // block_b, nc)

    def full2d(a):
        return pl.BlockSpec(a.shape, lambda b, c: (0, 0))

    kd = 8

    zm, zv = pl.pallas_call(
        functools.partial(_gru_chunk_kernel, nc=nc),
        out_shape=[jax.ShapeDtypeStruct((B_pad, latent_dim), jnp.float32),
                   jax.ShapeDtypeStruct((B_pad, latent_dim), jnp.float32)],
        grid=grid,
        in_specs=[
            pl.BlockSpec((cs, block_b, Din), lambda b, c: (nc - 1 - c, b, 0)),
            pl.BlockSpec((cs, block_b, 1), lambda b, c: (nc - 1 - c, b, 0)),
            pl.BlockSpec((1, block_b, 1),
                         lambda b, c: (jnp.maximum((nc - 1 - c) * cs - 1, 0),
                                       b, 0)),
            full2d(wihx),
            full2d(wiht),
            full2d(whh),
            full2d(bgi),
            full2d(bhn),
            full2d(wl),
            full2d(bl),
        ],
        out_specs=[pl.BlockSpec((block_b, latent_dim), lambda b, c: (b, 0)),
                   pl.BlockSpec((block_b, latent_dim), lambda b, c: (b, 0))],
        scratch_shapes=[
            pltpu.VMEM((block_b, HP), jnp.float32),
            pltpu.VMEM((cs, block_b, kd), jnp.float32),
            pltpu.VMEM((Din + HP, 2 * HP), jnp.float32),
            pltpu.VMEM((Din + kd, HP), jnp.float32),
            pltpu.VMEM((HP + kd, HP), jnp.float32),
        ],
        compiler_params=pltpu.CompilerParams(
            dimension_semantics=("parallel", "arbitrary"),
            vmem_limit_bytes=48 * 1024 * 1024),
    )(x, t, t, wihx, wiht, whh, bgi, bhn, wl, bl)

    return zm[:B], zv[:B]


# speedup vs baseline: 1.2709x; 1.2709x over previous
import functools

import jax
import jax.numpy as jnp
from jax import lax
from jax.experimental import pallas as pl
from jax.experimental.pallas import tpu as pltpu


def _sigmoid(a):
    return 0.5 * jnp.tanh(0.5 * a) + 0.5


def _gru_chunk_kernel(x_ref, ta_ref, tb_ref, wihx_ref, wiht_ref, whh_ref,
                      bgi_ref, bhn_ref, wl_ref, bl_ref, zm_ref, zv_ref,
                      h_ref, do_ref, wdt_ref, *, nc):
    c = pl.program_id(1)
    CS, BB, Din = x_ref.shape
    HP = whh_ref.shape[0]
    G3 = whh_ref.shape[1]
    KD = wdt_ref.shape[0] - Din

    @pl.when(c == 0)
    def _init():
        h_ref[...] = jnp.zeros_like(h_ref)
        wdt_ref[...] = jnp.concatenate(
            [wihx_ref[...], wiht_ref[...], bgi_ref[...],
             jnp.zeros((KD - 2, G3), jnp.float32)], axis=0)

    ta = ta_ref[...]
    d_rest = ta[:-1] - ta[1:]
    mask = jnp.where(c == nc - 1, 0.0, 1.0)
    d0 = (tb_ref[0:1] - ta[0:1]) * mask
    dt = jnp.concatenate([d0, d_rest], axis=0)
    ones = jnp.ones((CS, BB, 1), jnp.float32)
    zeros = jnp.zeros((CS, BB, KD - 2), jnp.float32)
    do_ref[...] = jnp.concatenate([dt, ones, zeros], axis=2)

    HB = BB // 2
    bhn = jnp.broadcast_to(bhn_ref[...], (HB, HP))

    def half(xh, dh, h):
        xd = jnp.concatenate([xh, dh], axis=1)
        gi = jnp.dot(xd, wdt_ref[...], preferred_element_type=jnp.float32)
        gh = jnp.dot(h, whh_ref[...], preferred_element_type=jnp.float32)
        r = _sigmoid(gi[:, :HP] + gh[:, :HP])
        z = _sigmoid(gi[:, HP:2 * HP] + gh[:, HP:2 * HP])
        n = jnp.tanh(gi[:, 2 * HP:] + r * (gh[:, 2 * HP:] + bhn))
        return (1.0 - z) * n + z * h

    def step(i, carry):
        h0, h1 = carry
        s = CS - 1 - i
        xs = x_ref[s]
        ds = do_ref[s]
        return (half(xs[:HB], ds[:HB], h0), half(xs[HB:], ds[HB:], h1))

    h0, h1 = lax.fori_loop(0, CS, step, (h_ref[:HB], h_ref[HB:]),
                           unroll=True)
    h_ref[:HB] = h0
    h_ref[HB:] = h1

    @pl.when(c == nc - 1)
    def _head():
        L = zm_ref.shape[1]
        z0 = (jnp.dot(h_ref[...], wl_ref[...], preferred_element_type=jnp.float32)
              + bl_ref[...])
        zm_ref[...] = z0[:, :L]
        zv_ref[...] = z0[:, L:]


def kernel(x, t, wihx, wiht, whh, bgi, bhn, wl, bl):
    S, B, Din = x.shape
    HP = whh.shape[0]
    G3 = whh.shape[1]
    L2 = wl.shape[1]
    latent_dim = L2 // 2

    x = x.astype(jnp.float32)
    t = t.astype(jnp.float32)

    B_pad = max(16, ((B + 15) // 16) * 16)
    block_b = min(B_pad, 256)
    block_b = max(16, (block_b // 16) * 16)
    B_pad = ((B_pad + block_b - 1) // block_b) * block_b
    if B_pad != B:
        x = jnp.pad(x, ((0, 0), (0, B_pad - B), (0, 0)))
        t = jnp.pad(t, ((0, 0), (0, B_pad - B), (0, 0)))

    cs = 1
    for cand in range(min(S, 50), 0, -1):
        if S % cand == 0:
            cs = cand
            break
    nc = S // cs

    grid = (B_pad // block_b, nc)

    def full2d(a):
        return pl.BlockSpec(a.shape, lambda b, c: (0, 0))

    kd = 8

    zm, zv = pl.pallas_call(
        functools.partial(_gru_chunk_kernel, nc=nc),
        out_shape=[jax.ShapeDtypeStruct((B_pad, latent_dim), jnp.float32),
                   jax.ShapeDtypeStruct((B_pad, latent_dim), jnp.float32)],
        grid=grid,
        in_specs=[
            pl.BlockSpec((cs, block_b, Din), lambda b, c: (nc - 1 - c, b, 0)),
            pl.BlockSpec((cs, block_b, 1), lambda b, c: (nc - 1 - c, b, 0)),
            pl.BlockSpec((1, block_b, 1),
                         lambda b, c: (jnp.maximum((nc - 1 - c) * cs - 1, 0),
                                       b, 0)),
            full2d(wihx),
            full2d(wiht),
            full2d(whh),
            full2d(bgi),
            full2d(bhn),
            full2d(wl),
            full2d(bl),
        ],
        out_specs=[pl.BlockSpec((block_b, latent_dim), lambda b, c: (b, 0)),
                   pl.BlockSpec((block_b, latent_dim), lambda b, c: (b, 0))],
        scratch_shapes=[
            pltpu.VMEM((block_b, HP), jnp.float32),
            pltpu.VMEM((cs, block_b, kd), jnp.float32),
            pltpu.VMEM((Din + kd, G3), jnp.float32),
        ],
        compiler_params=pltpu.CompilerParams(
            dimension_semantics=("parallel", "arbitrary"),
            vmem_limit_bytes=48 * 1024 * 1024),
    )(x, t, t, wihx, wiht, whh, bgi, bhn, wl, bl)

    return zm[:B], zv[:B]
